# baseline (device time: 18642 ns/iter reference)
import jax
import jax.numpy as jnp
from jax import lax
from jax.experimental import pallas as pl
from jax.experimental.pallas import tpu as pltpu

N_DEV = 4
N_CHUNK = 4


def kernel(x):
    x2 = x.reshape(x.shape[1], x.shape[2])
    m, n = x2.shape
    q = m // (2 * N_CHUNK)

    def body(x_ref, out_ref, comm_ref, send_sems, recv_sems):
        my = lax.axis_index("i")
        p1 = my ^ 1
        p2 = 3 - my

        barrier_sem = pltpu.get_barrier_semaphore()
        for nbr in [p1, p2]:
            pl.semaphore_signal(
                barrier_sem, inc=1,
                device_id=(nbr,), device_id_type=pl.DeviceIdType.MESH,
            )
        pl.semaphore_wait(barrier_sem, 2)

        n_half = 2 * N_CHUNK

        def mk(src, slot, dst_dev):
            return pltpu.make_async_remote_copy(
                src_ref=src,
                dst_ref=comm_ref.at[slot],
                send_sem=send_sems.at[slot],
                recv_sem=recv_sems.at[slot],
                device_id=(dst_dev,),
                device_id_type=pl.DeviceIdType.MESH,
            )

        r1 = []
        for c in range(n_half):
            dev = p1 if c < N_CHUNK else p2
            r1.append(mk(x_ref.at[pl.ds(c * q, q), :], c, dev))
        for rdma in r1:
            rdma.start()

        order = [c for pair in zip(range(N_CHUNK), range(N_CHUNK, n_half))
                 for c in pair]
        r2 = [None] * n_half
        for c in order:
            dev = p2 if c < N_CHUNK else p1
            r1[c].wait_recv()
            rows = pl.ds(c * q, q)
            out_ref[rows, :] = x_ref[rows, :] + comm_ref[c]
            r2[c] = mk(out_ref.at[rows, :], n_half + c, dev)
            r2[c].start()

        for c in order:
            r2[c].wait()
            rows = pl.ds(c * q, q)
            out_ref[rows, :] = out_ref[rows, :] + comm_ref[n_half + c]

        for rdma in r1:
            rdma.wait_send()

    return pl.pallas_call(
        body,
        out_shape=jax.ShapeDtypeStruct((m, n), x2.dtype),
        in_specs=[pl.BlockSpec(memory_space=pltpu.VMEM)],
        out_specs=pl.BlockSpec(memory_space=pltpu.VMEM),
        scratch_shapes=[
            pltpu.VMEM((4 * N_CHUNK, q, n), x2.dtype),
            pltpu.SemaphoreType.DMA((4 * N_CHUNK,)),
            pltpu.SemaphoreType.DMA((4 * N_CHUNK,)),
        ],
        compiler_params=pltpu.CompilerParams(collective_id=0),
    )(x2)


# device time: 18624 ns/iter; 1.0010x vs baseline; 1.0010x over previous
import jax
import jax.numpy as jnp
from jax import lax
from jax.experimental import pallas as pl
from jax.experimental.pallas import tpu as pltpu

N_DEV = 4
N_CHUNK = 4


def kernel(x):
    _, m, n = x.shape
    q = m // (2 * N_CHUNK)

    def body(x_ref, out_ref, comm_ref, send_sems, recv_sems):
        my = lax.axis_index("i")
        p1 = my ^ 1
        p2 = 3 - my

        barrier_sem = pltpu.get_barrier_semaphore()
        for nbr in [p1, p2]:
            pl.semaphore_signal(
                barrier_sem, inc=1,
                device_id=(nbr,), device_id_type=pl.DeviceIdType.MESH,
            )
        pl.semaphore_wait(barrier_sem, 2)

        n_half = 2 * N_CHUNK

        def mk(src, slot, dst_dev):
            return pltpu.make_async_remote_copy(
                src_ref=src,
                dst_ref=comm_ref.at[slot],
                send_sem=send_sems.at[slot],
                recv_sem=recv_sems.at[slot],
                device_id=(dst_dev,),
                device_id_type=pl.DeviceIdType.MESH,
            )

        r1 = []
        for c in range(n_half):
            dev = p1 if c < N_CHUNK else p2
            r1.append(mk(x_ref.at[0, pl.ds(c * q, q), :], c, dev))
        for rdma in r1:
            rdma.start()

        order = [c for pair in zip(range(N_CHUNK), range(N_CHUNK, n_half))
                 for c in pair]
        r2 = [None] * n_half
        for c in order:
            dev = p2 if c < N_CHUNK else p1
            r1[c].wait_recv()
            rows = pl.ds(c * q, q)
            out_ref[rows, :] = x_ref[0, rows, :] + comm_ref[c]
            r2[c] = mk(out_ref.at[rows, :], n_half + c, dev)
            r2[c].start()

        for c in order:
            r2[c].wait()
            rows = pl.ds(c * q, q)
            out_ref[rows, :] = out_ref[rows, :] + comm_ref[n_half + c]

        for rdma in r1:
            rdma.wait_send()

    return pl.pallas_call(
        body,
        out_shape=jax.ShapeDtypeStruct((m, n), x.dtype),
        in_specs=[pl.BlockSpec(memory_space=pltpu.VMEM)],
        out_specs=pl.BlockSpec(memory_space=pltpu.VMEM),
        scratch_shapes=[
            pltpu.VMEM((4 * N_CHUNK, q, n), x.dtype),
            pltpu.SemaphoreType.DMA((4 * N_CHUNK,)),
            pltpu.SemaphoreType.DMA((4 * N_CHUNK,)),
        ],
        compiler_params=pltpu.CompilerParams(collective_id=0),
    )(x)


# device time: 18564 ns/iter; 1.0042x vs baseline; 1.0032x over previous
import jax
import jax.numpy as jnp
from jax import lax
from jax.experimental import pallas as pl
from jax.experimental.pallas import tpu as pltpu

N_DEV = 4
N_CHUNK = 2


def kernel(x):
    _, m, n = x.shape
    q = m // (2 * N_CHUNK)

    def body(x_ref, out_ref, comm_ref, send_sems, recv_sems, acc_ref, copy_sems):
        my = lax.axis_index("i")
        p1 = my ^ 1
        p2 = 3 - my

        barrier_sem = pltpu.get_barrier_semaphore()
        for nbr in [p1, p2]:
            pl.semaphore_signal(
                barrier_sem, inc=1,
                device_id=(nbr,), device_id_type=pl.DeviceIdType.MESH,
            )
        pl.semaphore_wait(barrier_sem, 2)

        n_half = 2 * N_CHUNK

        def mk(src, slot, dst_dev):
            return pltpu.make_async_remote_copy(
                src_ref=src,
                dst_ref=comm_ref.at[slot],
                send_sem=send_sems.at[slot],
                recv_sem=recv_sems.at[slot],
                device_id=(dst_dev,),
                device_id_type=pl.DeviceIdType.MESH,
            )

        r1 = []
        for c in range(n_half):
            dev = p1 if c < N_CHUNK else p2
            r1.append(mk(x_ref.at[0, pl.ds(c * q, q), :], c, dev))
        for rdma in r1:
            rdma.start()

        order = [c for pair in zip(range(N_CHUNK), range(N_CHUNK, n_half))
                 for c in pair]
        r2 = [None] * n_half
        for c in order:
            dev = p2 if c < N_CHUNK else p1
            r1[c].wait_recv()
            rows = pl.ds(c * q, q)
            acc_ref[rows, :] = x_ref[0, rows, :] + comm_ref[c]
            r2[c] = mk(acc_ref.at[rows, :], n_half + c, dev)
            r2[c].start()

        copies = []
        for c in order:
            r2[c].wait()
            rows = pl.ds(c * q, q)
            acc_ref[rows, :] = acc_ref[rows, :] + comm_ref[n_half + c]
            cp = pltpu.make_async_copy(
                acc_ref.at[rows, :], out_ref.at[rows, :], copy_sems.at[c]
            )
            cp.start()
            copies.append(cp)
        for cp in copies:
            cp.wait()

        for rdma in r1:
            rdma.wait_send()

    return pl.pallas_call(
        body,
        out_shape=jax.ShapeDtypeStruct((m, n), x.dtype),
        in_specs=[pl.BlockSpec(memory_space=pltpu.VMEM)],
        out_specs=pl.BlockSpec(memory_space=pl.ANY),
        scratch_shapes=[
            pltpu.VMEM((4 * N_CHUNK, q, n), x.dtype),
            pltpu.SemaphoreType.DMA((4 * N_CHUNK,)),
            pltpu.SemaphoreType.DMA((4 * N_CHUNK,)),
            pltpu.VMEM((m, n), x.dtype),
            pltpu.SemaphoreType.DMA((2 * N_CHUNK,)),
        ],
        compiler_params=pltpu.CompilerParams(collective_id=0),
    )(x)
